# baseline (speedup 1.0000x reference)
import numpy as np
import jax
import jax.numpy as jnp
from jax.experimental import pallas as pl
from jax.experimental.pallas import tpu as pltpu

_THRESH = 0.6
_MIN_KEPT = 200000
_FACTOR = 8
_IGNORE = -1

_N, _C, _H, _W = 4, 19, 512, 512
_OH, _OW = 64, 64
_NDS = _N * _OH * _OW
_K = min(_NDS, _MIN_KEPT // (_FACTOR * _FACTOR)) - 1
_BH = 256
_NHB = _H // _BH
_DSB = _OH // _NHB

_INTERPRET = False


def _grid_coords(size, out):
    c = (np.arange(out) * (size - 1)).astype(np.float32) / np.float32(out - 1)
    lo = np.floor(c).astype(np.int32)
    hi = np.minimum(lo + 1, size - 1).astype(np.int32)
    frac = (c - lo.astype(np.float32)).astype(np.float32)
    near = np.clip(np.floor(c + 0.5).astype(np.int32), 0, size - 1)
    return lo, hi, frac, near


_H0, _H1, _FH, _IH = _grid_coords(_H, _OH)
_W0, _W1, _FW, _IW = _grid_coords(_W, _OW)

_RW = np.zeros((_NHB, _DSB, _BH), np.float32)
for _i in range(_OH):
    _hb, _il = _i // _DSB, _i % _DSB
    _RW[_hb, _il, _H0[_i] - _BH * _hb] += np.float32(1.0) - _FH[_i]
    _RW[_hb, _il, _H1[_i] - _BH * _hb] += _FH[_i]

_CW = np.zeros((_W, _OW), np.float32)
_CW[_W0, np.arange(_OW)] += np.float32(1.0) - _FW
_CW[_W1, np.arange(_OW)] += _FW

_JMAP = np.zeros(_W, np.int32)
_CVALID = np.zeros(_W, bool)
_JMAP[_W0] = np.arange(_OW)
_CVALID[_W0] = True
_JMAP[_W1] = np.arange(_OW)
_CVALID[_W1] = True
_ECOL = np.zeros((_OW, _W), np.float32)
_ECOL[_JMAP[_CVALID], np.nonzero(_CVALID)[0]] = 1.0
_EROW = np.zeros((_NHB, _BH, _DSB), np.float32)
for _i in range(_OH):
    _hb = _i // _DSB
    _EROW[_hb, _H0[_i] - _BH * _hb, _i % _DSB] = 1.0
    _EROW[_hb, _H1[_i] - _BH * _hb, _i % _DSB] = 1.0
_NSEL = np.zeros((_NHB, _DSB, _BH), np.float32)
for _i in range(_OH):
    _hb = _i // _DSB
    _NSEL[_hb, _i % _DSB, _IH[_i] - _BH * _hb] = 1.0
_NCOL = np.zeros((_W, _OW), np.float32)
_NCOL[_IW, np.arange(_OW)] = 1.0


def _dot(a, b):
    return jax.lax.dot_general(
        a, b, (((1,), (0,)), ((), ())), preferred_element_type=jnp.float32)


_NBLK = _N * _NHB


def _fused_kernel(x_ref, lbl_ref, nsel_ref, ncol_ref, erow_ref, ecol_ref,
                  rw_ref, cw_ref, out_ref,
                  pred_s, lbl_s, ds_s, thr_s):
    i = pl.program_id(0)

    @pl.when(i < _NBLK)
    def _stream():
        _stream_block(x_ref, lbl_ref, nsel_ref, ncol_ref, erow_ref,
                      ecol_ref, rw_ref, cw_ref, pred_s, lbl_s, ds_s, i)

    @pl.when(i == _NBLK)
    def _threshold():
        v = jax.lax.bitcast_convert_type(ds_s[...], jnp.int32)

        def body(_, carry):
            lo_b, hi_b = carry
            mid = lo_b + (hi_b - lo_b) // 2
            cnt = jnp.sum((v <= mid).astype(jnp.int32))
            ge = cnt >= (_K + 1)
            return (jnp.where(ge, lo_b, mid + 1), jnp.where(ge, mid, hi_b))

        lo_b, _hi = jax.lax.fori_loop(
            0, 31, body, (jnp.int32(0), jnp.int32(0x7F7FFFFF)))
        kth = jax.lax.bitcast_convert_type(lo_b, jnp.float32)
        thr_s[0] = jnp.where(kth > _THRESH, kth, jnp.float32(_THRESH))

    @pl.when(i >= _NBLK)
    def _mask():
        k = i - _NBLK
        t = thr_s[0]
        l = lbl_s[k]
        keep = (l >= 0) & (pred_s[k] <= t)
        out_ref[0] = jnp.where(keep, l, _IGNORE)


def _stream_block(x_ref, lbl_ref, nsel_ref, ncol_ref, erow_ref, ecol_ref,
                  rw_ref, cw_ref, pred_s, lbl_s, ds_s, blk):
    x = x_ref[0]
    l = lbl_ref[0]
    lblf = (l + 1).astype(jnp.float32)
    ldsb = _dot(_dot(nsel_ref[0], lblf), ncol_ref[...])
    l2f = _dot(erow_ref[0], _dot(ldsb, ecol_ref[...]))
    m = x[0]
    for c in range(1, _C):
        m = jnp.maximum(m, x[c])
    s = jnp.zeros_like(m)
    el = jnp.zeros_like(m)
    t2 = jnp.zeros_like(m)
    for c in range(_C):
        e = jnp.exp(x[c] - m)
        s = s + e
        el = jnp.where(l == c, e, el)
        t2 = jnp.where(l2f == np.float32(c + 1), e, t2)
    pred_s[blk] = el / s
    lbl_s[blk] = l
    v = t2 / s
    p = _dot(rw_ref[0], v)
    ds_s[blk] = _dot(p, cw_ref[...])


def kernel(predict, target):
    lbl32 = target.astype(jnp.int32)

    def _in_blk(i):
        j = jnp.minimum(i, _NBLK - 1)
        return (j // _NHB, j % _NHB, 0)

    def _in_blk_x(i):
        j = jnp.minimum(i, _NBLK - 1)
        return (j // _NHB, 0, j % _NHB, 0)

    def _in_blk_h(i):
        return (jnp.minimum(i, _NBLK - 1) % _NHB, 0, 0)

    def _out_blk(i):
        k = jnp.maximum(i - _NBLK, 0)
        return (k // _NHB, k % _NHB, 0)

    out = pl.pallas_call(
        _fused_kernel,
        grid=(2 * _NBLK,),
        in_specs=[
            pl.BlockSpec((1, _C, _BH, _W), _in_blk_x),
            pl.BlockSpec((1, _BH, _W), _in_blk),
            pl.BlockSpec((1, _DSB, _BH), _in_blk_h),
            pl.BlockSpec((_W, _OW), lambda i: (0, 0)),
            pl.BlockSpec((1, _BH, _DSB), _in_blk_h),
            pl.BlockSpec((_OW, _W), lambda i: (0, 0)),
            pl.BlockSpec((1, _DSB, _BH), _in_blk_h),
            pl.BlockSpec((_W, _OW), lambda i: (0, 0)),
        ],
        out_specs=pl.BlockSpec((1, _BH, _W), _out_blk),
        out_shape=jax.ShapeDtypeStruct((_N, _H, _W), jnp.int32),
        scratch_shapes=[
            pltpu.VMEM((_NBLK, _BH, _W), jnp.float32),
            pltpu.VMEM((_NBLK, _BH, _W), jnp.int32),
            pltpu.VMEM((_NBLK, _DSB, _OW), jnp.float32),
            pltpu.SMEM((1,), jnp.float32),
        ],
        interpret=_INTERPRET,
    )(predict, lbl32, jnp.asarray(_NSEL), jnp.asarray(_NCOL),
      jnp.asarray(_EROW), jnp.asarray(_ECOL),
      jnp.asarray(_RW), jnp.asarray(_CW))

    return out.astype(jnp.int64)


# speedup vs baseline: 1.0298x; 1.0298x over previous
import numpy as np
import jax
import jax.numpy as jnp
from jax.experimental import pallas as pl
from jax.experimental.pallas import tpu as pltpu

_THRESH = 0.6
_MIN_KEPT = 200000
_FACTOR = 8
_IGNORE = -1

_N, _C, _H, _W = 4, 19, 512, 512
_OH, _OW = 64, 64
_NDS = _N * _OH * _OW
_K = min(_NDS, _MIN_KEPT // (_FACTOR * _FACTOR)) - 1
_BH = 512
_NHB = _H // _BH
_DSB = _OH // _NHB

def _grid_coords(size, out):
    c = (np.arange(out) * (size - 1)).astype(np.float32) / np.float32(out - 1)
    lo = np.floor(c).astype(np.int32)
    hi = np.minimum(lo + 1, size - 1).astype(np.int32)
    frac = (c - lo.astype(np.float32)).astype(np.float32)
    near = np.clip(np.floor(c + 0.5).astype(np.int32), 0, size - 1)
    return lo, hi, frac, near


_H0, _H1, _FH, _IH = _grid_coords(_H, _OH)
_W0, _W1, _FW, _IW = _grid_coords(_W, _OW)

_RW = np.zeros((_NHB, _DSB, _BH), np.float32)
for _i in range(_OH):
    _hb, _il = _i // _DSB, _i % _DSB
    _RW[_hb, _il, _H0[_i] - _BH * _hb] += np.float32(1.0) - _FH[_i]
    _RW[_hb, _il, _H1[_i] - _BH * _hb] += _FH[_i]

_CW = np.zeros((_W, _OW), np.float32)
_CW[_W0, np.arange(_OW)] += np.float32(1.0) - _FW
_CW[_W1, np.arange(_OW)] += _FW

_JMAP = np.zeros(_W, np.int32)
_CVALID = np.zeros(_W, bool)
_JMAP[_W0] = np.arange(_OW)
_CVALID[_W0] = True
_JMAP[_W1] = np.arange(_OW)
_CVALID[_W1] = True
_ECOL = np.zeros((_OW, _W), np.float32)
_ECOL[_JMAP[_CVALID], np.nonzero(_CVALID)[0]] = 1.0
_EROW = np.zeros((_NHB, _BH, _DSB), np.float32)
for _i in range(_OH):
    _hb = _i // _DSB
    _EROW[_hb, _H0[_i] - _BH * _hb, _i % _DSB] = 1.0
    _EROW[_hb, _H1[_i] - _BH * _hb, _i % _DSB] = 1.0
_NSEL = np.zeros((_NHB, _DSB, _BH), np.float32)
for _i in range(_OH):
    _hb = _i // _DSB
    _NSEL[_hb, _i % _DSB, _IH[_i] - _BH * _hb] = 1.0
_NCOL = np.zeros((_W, _OW), np.float32)
_NCOL[_IW, np.arange(_OW)] = 1.0


def _dot(a, b):
    return jax.lax.dot_general(
        a, b, (((1,), (0,)), ((), ())), preferred_element_type=jnp.float32)


_NBLK = _N * _NHB


def _fused_kernel(x_ref, lbl_ref, nsel_ref, ncol_ref, erow_ref, ecol_ref,
                  rw_ref, cw_ref, out_ref,
                  pred_s, lbl_s, ds_s, thr_s):
    i = pl.program_id(0)

    @pl.when(i < _NBLK)
    def _stream():
        _stream_block(x_ref, lbl_ref, nsel_ref, ncol_ref, erow_ref,
                      ecol_ref, rw_ref, cw_ref, pred_s, lbl_s, ds_s, i)

    @pl.when(i == _NBLK)
    def _threshold():
        v = jax.lax.bitcast_convert_type(ds_s[...], jnp.int32)

        def body(_, carry):
            lo_b, hi_b = carry
            mid = lo_b + (hi_b - lo_b) // 2
            cnt = jnp.sum((v <= mid).astype(jnp.int32))
            ge = cnt >= (_K + 1)
            return (jnp.where(ge, lo_b, mid + 1), jnp.where(ge, mid, hi_b))

        lo_b, _hi = jax.lax.fori_loop(
            0, 31, body, (jnp.int32(0), jnp.int32(0x7F7FFFFF)))
        kth = jax.lax.bitcast_convert_type(lo_b, jnp.float32)
        thr_s[0] = jnp.where(kth > _THRESH, kth, jnp.float32(_THRESH))

    @pl.when(i >= _NBLK)
    def _mask():
        k = i - _NBLK
        t = thr_s[0]
        l = lbl_s[k]
        keep = (l >= 0) & (pred_s[k] <= t)
        out_ref[0] = jnp.where(keep, l, _IGNORE)


def _stream_block(x_ref, lbl_ref, nsel_ref, ncol_ref, erow_ref, ecol_ref,
                  rw_ref, cw_ref, pred_s, lbl_s, ds_s, blk):
    x = x_ref[0]
    l = lbl_ref[0]
    lblf = (l + 1).astype(jnp.float32)
    ldsb = _dot(_dot(nsel_ref[0], lblf), ncol_ref[...])
    l2f = _dot(erow_ref[0], _dot(ldsb, ecol_ref[...]))
    m = x[0]
    for c in range(1, _C):
        m = jnp.maximum(m, x[c])
    s = jnp.zeros_like(m)
    el = jnp.zeros_like(m)
    t2 = jnp.zeros_like(m)
    for c in range(_C):
        e = jnp.exp(x[c] - m)
        s = s + e
        el = jnp.where(l == c, e, el)
        t2 = jnp.where(l2f == np.float32(c + 1), e, t2)
    pred_s[blk] = el / s
    lbl_s[blk] = l
    v = t2 / s
    p = _dot(rw_ref[0], v)
    ds_s[blk] = _dot(p, cw_ref[...])


def kernel(predict, target):
    lbl32 = target.astype(jnp.int32)

    def _in_blk(i):
        j = jnp.minimum(i, _NBLK - 1)
        return (j // _NHB, j % _NHB, 0)

    def _in_blk_x(i):
        j = jnp.minimum(i, _NBLK - 1)
        return (j // _NHB, 0, j % _NHB, 0)

    def _in_blk_h(i):
        return (jnp.minimum(i, _NBLK - 1) % _NHB, 0, 0)

    def _out_blk(i):
        k = jnp.maximum(i - _NBLK, 0)
        return (k // _NHB, k % _NHB, 0)

    out = pl.pallas_call(
        _fused_kernel,
        grid=(2 * _NBLK,),
        in_specs=[
            pl.BlockSpec((1, _C, _BH, _W), _in_blk_x),
            pl.BlockSpec((1, _BH, _W), _in_blk),
            pl.BlockSpec((1, _DSB, _BH), _in_blk_h),
            pl.BlockSpec((_W, _OW), lambda i: (0, 0)),
            pl.BlockSpec((1, _BH, _DSB), _in_blk_h),
            pl.BlockSpec((_OW, _W), lambda i: (0, 0)),
            pl.BlockSpec((1, _DSB, _BH), _in_blk_h),
            pl.BlockSpec((_W, _OW), lambda i: (0, 0)),
        ],
        out_specs=pl.BlockSpec((1, _BH, _W), _out_blk),
        out_shape=jax.ShapeDtypeStruct((_N, _H, _W), jnp.int32),
        scratch_shapes=[
            pltpu.VMEM((_NBLK, _BH, _W), jnp.float32),
            pltpu.VMEM((_NBLK, _BH, _W), jnp.int32),
            pltpu.VMEM((_NBLK, _DSB, _OW), jnp.float32),
            pltpu.SMEM((1,), jnp.float32),
        ],
    )(predict, lbl32, jnp.asarray(_NSEL), jnp.asarray(_NCOL),
      jnp.asarray(_EROW), jnp.asarray(_ECOL),
      jnp.asarray(_RW), jnp.asarray(_CW))

    return out.astype(jnp.int64)
